# baseline (device time: 1038184 ns/iter reference)
import functools

import jax
import jax.numpy as jnp
from jax import lax
from jax.experimental import pallas as pl
from jax.experimental.pallas import tpu as pltpu

N_DEV = 16
CAPACITY = 204
CAP_LOCAL = 48


def _ring_allgather_small(r2d):
    m, n = r2d.shape

    def body(x_ref, out_ref, send_sems, recv_sems):
        my = lax.axis_index("i")
        left = (my - 1) % N_DEV
        right = (my + 1) % N_DEV

        barrier = pltpu.get_barrier_semaphore()
        for nbr in (left, right):
            pl.semaphore_signal(barrier, inc=1, device_id=(nbr,),
                                device_id_type=pl.DeviceIdType.MESH)
        pl.semaphore_wait(barrier, 2)

        out_ref[pl.ds(my * m, m), :] = x_ref[:, :]
        for h in range(N_DEV - 1):
            c = (my - h) % N_DEV
            sl = pl.ds(c * m, m)
            rdma = pltpu.make_async_remote_copy(
                src_ref=out_ref.at[sl],
                dst_ref=out_ref.at[sl],
                send_sem=send_sems.at[h],
                recv_sem=recv_sems.at[h],
                device_id=(right,),
                device_id_type=pl.DeviceIdType.MESH,
            )
            rdma.start()
            rdma.wait()

        @functools.partial(pl.run_scoped, sem=pltpu.SemaphoreType.REGULAR)
        def _(sem):
            for nbr in (left, right):
                pl.semaphore_signal(sem, inc=1, device_id=(nbr,),
                                    device_id_type=pl.DeviceIdType.MESH)
            pl.semaphore_wait(sem, 2)

    return pl.pallas_call(
        body,
        out_shape=jax.ShapeDtypeStruct((N_DEV * m, n), r2d.dtype),
        in_specs=[pl.BlockSpec(memory_space=pltpu.VMEM)],
        out_specs=pl.BlockSpec(memory_space=pltpu.VMEM),
        scratch_shapes=[
            pltpu.SemaphoreType.DMA((N_DEV - 1,)),
            pltpu.SemaphoreType.DMA((N_DEV - 1,)),
        ],
        compiler_params=pltpu.CompilerParams(collective_id=0),
    )(r2d)


N_HOP_R = N_DEV // 2
N_HOP_L = N_DEV - 1 - N_HOP_R


def _ring_allgather_w(w):
    e_per, d, f = w.shape

    def body(w_ref, out_ref, copy_sem, send_r, recv_r, send_l, recv_l):
        my = lax.axis_index("i")
        left = (my - 1) % N_DEV
        right = (my + 1) % N_DEV

        cp = pltpu.make_async_copy(
            w_ref, out_ref.at[pl.ds(my * e_per, e_per)], copy_sem)
        cp.start()

        barrier = pltpu.get_barrier_semaphore()
        for nbr in (left, right):
            pl.semaphore_signal(barrier, inc=1, device_id=(nbr,),
                                device_id_type=pl.DeviceIdType.MESH)
        pl.semaphore_wait(barrier, 2)
        cp.wait()

        for h in range(N_HOP_R):
            c_r = (my - h) % N_DEV
            sl_r = pl.ds(c_r * e_per, e_per)
            rdma_r = pltpu.make_async_remote_copy(
                src_ref=out_ref.at[sl_r],
                dst_ref=out_ref.at[sl_r],
                send_sem=send_r.at[h],
                recv_sem=recv_r.at[h],
                device_id=(right,),
                device_id_type=pl.DeviceIdType.MESH,
            )
            rdma_r.start()
            if h < N_HOP_L:
                c_l = (my + h) % N_DEV
                sl_l = pl.ds(c_l * e_per, e_per)
                rdma_l = pltpu.make_async_remote_copy(
                    src_ref=out_ref.at[sl_l],
                    dst_ref=out_ref.at[sl_l],
                    send_sem=send_l.at[h],
                    recv_sem=recv_l.at[h],
                    device_id=(left,),
                    device_id_type=pl.DeviceIdType.MESH,
                )
                rdma_l.start()
            rdma_r.wait()
            if h < N_HOP_L:
                rdma_l.wait()

        @functools.partial(pl.run_scoped, sem=pltpu.SemaphoreType.REGULAR)
        def _(sem):
            for nbr in (left, right):
                pl.semaphore_signal(sem, inc=1, device_id=(nbr,),
                                    device_id_type=pl.DeviceIdType.MESH)
            pl.semaphore_wait(sem, 2)

    return pl.pallas_call(
        body,
        out_shape=jax.ShapeDtypeStruct((N_DEV * e_per, d, f), w.dtype),
        in_specs=[pl.BlockSpec(memory_space=pltpu.MemorySpace.HBM)],
        out_specs=pl.BlockSpec(memory_space=pltpu.MemorySpace.HBM),
        scratch_shapes=[
            pltpu.SemaphoreType.DMA,
            pltpu.SemaphoreType.DMA((N_HOP_R,)),
            pltpu.SemaphoreType.DMA((N_HOP_R,)),
            pltpu.SemaphoreType.DMA((N_HOP_L,)),
            pltpu.SemaphoreType.DMA((N_HOP_L,)),
        ],
        compiler_params=pltpu.CompilerParams(collective_id=1),
    )(w)


T_BLK = 1024
ROWS = T_BLK // 128


def _rank_pos_kernel(g_route2d, n_tok_local, n_exp):
    n_rows, _ = g_route2d.shape
    n_blocks = (n_rows * 128) // T_BLK
    blocks_per_dev = n_tok_local // T_BLK

    def body(r_ref, acc_ref, pos_ref, counts, counts2):
        b = pl.program_id(0)

        @pl.when(b == 0)
        def _():
            counts[0, :] = jnp.zeros((n_exp,), jnp.float32)

        @pl.when(b % blocks_per_dev == 0)
        def _():
            counts2[0, :] = jnp.zeros((n_exp,), jnp.float32)

        route3 = r_ref[...][:, :, None]
        e_iota = lax.broadcasted_iota(jnp.int32, (ROWS, 128, n_exp), 2)
        oh3 = (route3 == e_iota).astype(jnp.float32)
        oh_b = oh3.reshape(T_BLK, n_exp).astype(jnp.bfloat16)

        r_iota = lax.broadcasted_iota(jnp.int32, (T_BLK, T_BLK), 0)
        c_iota = lax.broadcasted_iota(jnp.int32, (T_BLK, T_BLK), 1)
        tril = (r_iota >= c_iota).astype(jnp.bfloat16)

        cs = (jnp.dot(tril, oh_b, preferred_element_type=jnp.float32)
              + counts[0, :][None, :])
        cs3 = cs.reshape(ROWS, 128, n_exp)
        cnt = jnp.sum(cs3 * oh3, axis=2)
        acc = (cnt <= CAPACITY).astype(jnp.float32)

        oh_acc = (oh3 * acc[:, :, None]).reshape(T_BLK, n_exp).astype(
            jnp.bfloat16)
        cs2 = (jnp.dot(tril, oh_acc, preferred_element_type=jnp.float32)
               + counts2[0, :][None, :])
        cs2_3 = cs2.reshape(ROWS, 128, n_exp)
        pos = jnp.sum(cs2_3 * oh3, axis=2) - 1.0

        counts[0, :] = cs[T_BLK - 1, :]
        counts2[0, :] = cs2[T_BLK - 1, :]

        acc_ref[...] = acc.astype(jnp.int32)
        pos_ref[...] = pos.astype(jnp.int32)

    acc2d, pos2d = pl.pallas_call(
        body,
        grid=(n_blocks,),
        in_specs=[pl.BlockSpec((ROWS, 128), lambda b: (b, 0))],
        out_specs=[
            pl.BlockSpec((ROWS, 128), lambda b: (b, 0)),
            pl.BlockSpec((ROWS, 128), lambda b: (b, 0)),
        ],
        out_shape=[
            jax.ShapeDtypeStruct((n_rows, 128), jnp.int32),
            jax.ShapeDtypeStruct((n_rows, 128), jnp.int32),
        ],
        scratch_shapes=[
            pltpu.VMEM((1, n_exp), jnp.float32),
            pltpu.VMEM((1, n_exp), jnp.float32),
        ],
    )(g_route2d)
    return acc2d, pos2d


def _grouped_matmul(x_grp, w_full):
    s, d = x_grp.shape
    n_exp, _, f = w_full.shape
    cap = s // n_exp

    def body(x_ref, w_ref, o_ref):
        o_ref[:, :] = jnp.dot(x_ref[:, :].astype(w_ref.dtype), w_ref[0],
                              preferred_element_type=jnp.float32)

    return pl.pallas_call(
        body,
        grid=(n_exp,),
        in_specs=[
            pl.BlockSpec((cap, d), lambda e: (e, 0)),
            pl.BlockSpec((1, d, f), lambda e: (e, 0, 0)),
        ],
        out_specs=pl.BlockSpec((cap, f), lambda e: (e, 0)),
        out_shape=jax.ShapeDtypeStruct((s, f), jnp.float32),
    )(x_grp, w_full)


DISP_BLK = 512


def _dispatch_gather(slot, x, n_slots):
    _, n_tok = slot.shape
    _, d = x.shape

    def body(s_ref, x_ref, o_ref):
        k = pl.program_id(0)
        s_iota = (lax.broadcasted_iota(jnp.int32, (DISP_BLK, n_tok), 0)
                  + k * DISP_BLK)
        oh = (s_iota == s_ref[...]).astype(jnp.bfloat16)
        o_ref[...] = jnp.dot(oh, x_ref[...].astype(jnp.bfloat16),
                             preferred_element_type=jnp.float32)

    return pl.pallas_call(
        body,
        grid=(n_slots // DISP_BLK,),
        in_specs=[
            pl.BlockSpec((1, n_tok), lambda k: (0, 0)),
            pl.BlockSpec((n_tok, d), lambda k: (0, 0)),
        ],
        out_specs=pl.BlockSpec((DISP_BLK, d), lambda k: (k, 0)),
        out_shape=jax.ShapeDtypeStruct((n_slots, d), jnp.float32),
    )(slot, x)


def _combine_scatter(slot, y_grp, n_tok):
    n_slots, f = y_grp.shape

    def body(s_ref, y_ref, o_ref):
        k = pl.program_id(0)

        @pl.when(k == 0)
        def _():
            o_ref[...] = jnp.zeros_like(o_ref)

        s_iota = (lax.broadcasted_iota(jnp.int32, (DISP_BLK, n_tok), 0)
                  + k * DISP_BLK)
        oh = (s_iota == s_ref[...]).astype(jnp.bfloat16)
        o_ref[...] += lax.dot_general(
            oh, y_ref[...].astype(jnp.bfloat16), (((0,), (0,)), ((), ())),
            preferred_element_type=jnp.float32)

    return pl.pallas_call(
        body,
        grid=(n_slots // DISP_BLK,),
        in_specs=[
            pl.BlockSpec((1, n_tok), lambda k: (0, 0)),
            pl.BlockSpec((DISP_BLK, f), lambda k: (k, 0)),
        ],
        out_specs=pl.BlockSpec((n_tok, f), lambda k: (0, 0)),
        out_shape=jax.ShapeDtypeStruct((n_tok, f), jnp.float32),
    )(slot, y_grp)


def kernel(x, router_W, route_idx, expert_W):
    del router_W
    n_tok, d_model = x.shape
    e_per, _, d_ff = expert_W.shape
    n_exp = N_DEV * e_per
    n_tok_global = N_DEV * n_tok

    my = lax.axis_index("i")

    r2d = route_idx.reshape(n_tok // 128, 128)
    g_route = _ring_allgather_small(r2d).reshape(n_tok_global)

    w_full = _ring_allgather_w(expert_W.astype(jnp.bfloat16))

    acc2d, pos2d = _rank_pos_kernel(g_route.reshape(-1, 128), n_tok, n_exp)
    acc_mine = lax.dynamic_slice(acc2d.reshape(n_tok_global),
                                 (my * n_tok,), (n_tok,))
    pos_mine = lax.dynamic_slice(pos2d.reshape(n_tok_global),
                                 (my * n_tok,), (n_tok,))

    e_mine = route_idx[:, 0].astype(jnp.int32)
    dump = n_exp * CAP_LOCAL
    valid = (acc_mine == 1) & (pos_mine < CAP_LOCAL)
    slot = jnp.where(valid, e_mine * CAP_LOCAL + pos_mine, dump)
    slot2d = slot.reshape(1, n_tok)

    x_grp = _dispatch_gather(slot2d, x, dump)
    out_grp = _grouped_matmul(x_grp, w_full)
    out = _combine_scatter(slot2d, out_grp, n_tok)
    return out


# device time: 952288 ns/iter; 1.0902x vs baseline; 1.0902x over previous
import functools

import jax
import jax.numpy as jnp
from jax import lax
from jax.experimental import pallas as pl
from jax.experimental.pallas import tpu as pltpu

N_DEV = 16
CAPACITY = 204
CAP_LOCAL = 48

T_BLK = 1024
ROWS = T_BLK // 128
DISP_BLK = 512
N_HOP_R = N_DEV // 2
N_HOP_L = N_DEV - 1 - N_HOP_R


def _ring_allgather_small(r2d):
    m, n = r2d.shape

    def body(x_ref, out_ref, send_sems, recv_sems):
        my = lax.axis_index("i")
        left = (my - 1) % N_DEV
        right = (my + 1) % N_DEV

        barrier = pltpu.get_barrier_semaphore()
        for nbr in (left, right):
            pl.semaphore_signal(barrier, inc=1, device_id=(nbr,),
                                device_id_type=pl.DeviceIdType.MESH)
        pl.semaphore_wait(barrier, 2)

        out_ref[pl.ds(my * m, m), :] = x_ref[:, :]
        for h in range(N_DEV - 1):
            c = (my - h) % N_DEV
            sl = pl.ds(c * m, m)
            rdma = pltpu.make_async_remote_copy(
                src_ref=out_ref.at[sl],
                dst_ref=out_ref.at[sl],
                send_sem=send_sems.at[h],
                recv_sem=recv_sems.at[h],
                device_id=(right,),
                device_id_type=pl.DeviceIdType.MESH,
            )
            rdma.start()
            rdma.wait()

        @functools.partial(pl.run_scoped, sem=pltpu.SemaphoreType.REGULAR)
        def _(sem):
            for nbr in (left, right):
                pl.semaphore_signal(sem, inc=1, device_id=(nbr,),
                                    device_id_type=pl.DeviceIdType.MESH)
            pl.semaphore_wait(sem, 2)

    return pl.pallas_call(
        body,
        out_shape=jax.ShapeDtypeStruct((N_DEV * m, n), r2d.dtype),
        in_specs=[pl.BlockSpec(memory_space=pltpu.VMEM)],
        out_specs=pl.BlockSpec(memory_space=pltpu.VMEM),
        scratch_shapes=[
            pltpu.SemaphoreType.DMA((N_DEV - 1,)),
            pltpu.SemaphoreType.DMA((N_DEV - 1,)),
        ],
        compiler_params=pltpu.CompilerParams(collective_id=0),
    )(r2d)


def _fused_wgather_rank_dispatch(route_g, e2d, x, w):
    n_rows, _ = route_g.shape
    n_glob = n_rows * 128
    m_loc, _ = e2d.shape
    n_tok, d = x.shape
    e_per, _, f = w.shape
    n_exp = N_DEV * e_per
    dump = n_exp * CAP_LOCAL
    n_blocks = n_glob // T_BLK
    blocks_per_dev = n_tok // T_BLK

    def body(route_ref, e_ref, x_ref, w_ref, wout_ref, xg_ref, slot_ref,
             copy_sem, send_r, recv_r, send_l, recv_l,
             acc_buf, pos_buf, counts, counts2):
        my = lax.axis_index("i")
        left = (my - 1) % N_DEV
        right = (my + 1) % N_DEV

        cp = pltpu.make_async_copy(
            w_ref, wout_ref.at[pl.ds(my * e_per, e_per)], copy_sem)
        cp.start()

        barrier = pltpu.get_barrier_semaphore()
        for nbr in (left, right):
            pl.semaphore_signal(barrier, inc=1, device_id=(nbr,),
                                device_id_type=pl.DeviceIdType.MESH)
        pl.semaphore_wait(barrier, 2)
        cp.wait()

        r_iota = lax.broadcasted_iota(jnp.int32, (T_BLK, T_BLK), 0)
        c_iota = lax.broadcasted_iota(jnp.int32, (T_BLK, T_BLK), 1)
        tril = (r_iota >= c_iota).astype(jnp.bfloat16)

        def rank_block(b):
            if b % blocks_per_dev == 0:
                counts2[0, :] = jnp.zeros((n_exp,), jnp.float32)
            route3 = route_ref[pl.ds(b * ROWS, ROWS), :][:, :, None]
            e_iota = lax.broadcasted_iota(jnp.int32, (ROWS, 128, n_exp), 2)
            oh3 = (route3 == e_iota).astype(jnp.float32)
            oh_b = oh3.reshape(T_BLK, n_exp).astype(jnp.bfloat16)
            cs = (jnp.dot(tril, oh_b, preferred_element_type=jnp.float32)
                  + counts[0, :][None, :])
            cs3 = cs.reshape(ROWS, 128, n_exp)
            cnt = jnp.sum(cs3 * oh3, axis=2)
            acc = (cnt <= CAPACITY).astype(jnp.float32)
            oh_acc = (oh3 * acc[:, :, None]).reshape(T_BLK, n_exp).astype(
                jnp.bfloat16)
            cs2 = (jnp.dot(tril, oh_acc, preferred_element_type=jnp.float32)
                   + counts2[0, :][None, :])
            cs2_3 = cs2.reshape(ROWS, 128, n_exp)
            pos = jnp.sum(cs2_3 * oh3, axis=2) - 1.0
            counts[0, :] = cs[T_BLK - 1, :]
            counts2[0, :] = cs2[T_BLK - 1, :]
            acc_buf[pl.ds(b * ROWS, ROWS), :] = acc.astype(jnp.int32)
            pos_buf[pl.ds(b * ROWS, ROWS), :] = pos.astype(jnp.int32)

        def dispatch_phase():
            acc16 = acc_buf[pl.ds(my * m_loc, m_loc), :]
            pos16 = pos_buf[pl.ds(my * m_loc, m_loc), :]
            e16 = e_ref[...]
            valid = (acc16 == 1) & (pos16 < CAP_LOCAL)
            slot16 = jnp.where(valid, e16 * CAP_LOCAL + pos16, dump)
            slot_ref[...] = slot16
            slot_flat = jnp.concatenate(
                [slot16[j:j + 1, :] for j in range(m_loc)], axis=1)
            xb = x_ref[...].astype(jnp.bfloat16)
            for k in range(dump // DISP_BLK):
                s_iota = (lax.broadcasted_iota(
                    jnp.int32, (DISP_BLK, n_tok), 0) + k * DISP_BLK)
                oh = (s_iota == slot_flat).astype(jnp.bfloat16)
                xg_ref[pl.ds(k * DISP_BLK, DISP_BLK), :] = jnp.dot(
                    oh, xb, preferred_element_type=jnp.float32)

        def compute_phase(h):
            if h == 0:
                counts[0, :] = jnp.zeros((n_exp,), jnp.float32)
                for b in range(n_blocks // 2):
                    rank_block(b)
            elif h == 1:
                for b in range(n_blocks // 2, n_blocks):
                    rank_block(b)
            elif h == 2:
                dispatch_phase()

        for h in range(N_HOP_R):
            c_r = (my - h) % N_DEV
            sl_r = pl.ds(c_r * e_per, e_per)
            rdma_r = pltpu.make_async_remote_copy(
                src_ref=wout_ref.at[sl_r],
                dst_ref=wout_ref.at[sl_r],
                send_sem=send_r.at[h],
                recv_sem=recv_r.at[h],
                device_id=(right,),
                device_id_type=pl.DeviceIdType.MESH,
            )
            rdma_r.start()
            if h < N_HOP_L:
                c_l = (my + h) % N_DEV
                sl_l = pl.ds(c_l * e_per, e_per)
                rdma_l = pltpu.make_async_remote_copy(
                    src_ref=wout_ref.at[sl_l],
                    dst_ref=wout_ref.at[sl_l],
                    send_sem=send_l.at[h],
                    recv_sem=recv_l.at[h],
                    device_id=(left,),
                    device_id_type=pl.DeviceIdType.MESH,
                )
                rdma_l.start()
            compute_phase(h)
            rdma_r.wait()
            if h < N_HOP_L:
                rdma_l.wait()

        @functools.partial(pl.run_scoped, sem=pltpu.SemaphoreType.REGULAR)
        def _(sem):
            for nbr in (left, right):
                pl.semaphore_signal(sem, inc=1, device_id=(nbr,),
                                    device_id_type=pl.DeviceIdType.MESH)
            pl.semaphore_wait(sem, 2)

    return pl.pallas_call(
        body,
        out_shape=[
            jax.ShapeDtypeStruct((N_DEV * e_per, d, f), w.dtype),
            jax.ShapeDtypeStruct((dump, d), jnp.float32),
            jax.ShapeDtypeStruct((m_loc, 128), jnp.int32),
        ],
        in_specs=[
            pl.BlockSpec(memory_space=pltpu.VMEM),
            pl.BlockSpec(memory_space=pltpu.VMEM),
            pl.BlockSpec(memory_space=pltpu.VMEM),
            pl.BlockSpec(memory_space=pltpu.MemorySpace.HBM),
        ],
        out_specs=[
            pl.BlockSpec(memory_space=pltpu.MemorySpace.HBM),
            pl.BlockSpec(memory_space=pltpu.VMEM),
            pl.BlockSpec(memory_space=pltpu.VMEM),
        ],
        scratch_shapes=[
            pltpu.SemaphoreType.DMA,
            pltpu.SemaphoreType.DMA((N_HOP_R,)),
            pltpu.SemaphoreType.DMA((N_HOP_R,)),
            pltpu.SemaphoreType.DMA((N_HOP_L,)),
            pltpu.SemaphoreType.DMA((N_HOP_L,)),
            pltpu.VMEM((n_rows, 128), jnp.int32),
            pltpu.VMEM((n_rows, 128), jnp.int32),
            pltpu.VMEM((1, n_exp), jnp.float32),
            pltpu.VMEM((1, n_exp), jnp.float32),
        ],
        compiler_params=pltpu.CompilerParams(collective_id=1),
    )(route_g, e2d, x, w)


def _grouped_matmul(x_grp, w_full):
    s, d = x_grp.shape
    n_exp, _, f = w_full.shape
    cap = s // n_exp

    def body(x_ref, w_ref, o_ref):
        o_ref[:, :] = jnp.dot(x_ref[:, :].astype(w_ref.dtype), w_ref[0],
                              preferred_element_type=jnp.float32)

    return pl.pallas_call(
        body,
        grid=(n_exp,),
        in_specs=[
            pl.BlockSpec((cap, d), lambda e: (e, 0)),
            pl.BlockSpec((1, d, f), lambda e: (e, 0, 0)),
        ],
        out_specs=pl.BlockSpec((cap, f), lambda e: (e, 0)),
        out_shape=jax.ShapeDtypeStruct((s, f), jnp.float32),
    )(x_grp, w_full)


def _combine_scatter(slot, y_grp, n_tok):
    n_slots, f = y_grp.shape

    def body(s_ref, y_ref, o_ref):
        k = pl.program_id(0)

        @pl.when(k == 0)
        def _():
            o_ref[...] = jnp.zeros_like(o_ref)

        s_iota = (lax.broadcasted_iota(jnp.int32, (DISP_BLK, n_tok), 0)
                  + k * DISP_BLK)
        oh = (s_iota == s_ref[...]).astype(jnp.bfloat16)
        o_ref[...] += lax.dot_general(
            oh, y_ref[...].astype(jnp.bfloat16), (((0,), (0,)), ((), ())),
            preferred_element_type=jnp.float32)

    return pl.pallas_call(
        body,
        grid=(n_slots // DISP_BLK,),
        in_specs=[
            pl.BlockSpec((1, n_tok), lambda k: (0, 0)),
            pl.BlockSpec((DISP_BLK, f), lambda k: (k, 0)),
        ],
        out_specs=pl.BlockSpec((n_tok, f), lambda k: (0, 0)),
        out_shape=jax.ShapeDtypeStruct((n_tok, f), jnp.float32),
    )(slot, y_grp)


def kernel(x, router_W, route_idx, expert_W):
    del router_W
    n_tok, d_model = x.shape
    e_per, _, d_ff = expert_W.shape

    r2d = route_idx.reshape(n_tok // 128, 128)
    route_g = _ring_allgather_small(r2d)

    w_full, x_grp, slot16 = _fused_wgather_rank_dispatch(
        route_g, r2d, x, expert_W.astype(jnp.bfloat16))

    out_grp = _grouped_matmul(x_grp, w_full)
    out = _combine_scatter(slot16.reshape(1, n_tok), out_grp, n_tok)
    return out


# device time: 916542 ns/iter; 1.1327x vs baseline; 1.0390x over previous
import functools

import jax
import jax.numpy as jnp
from jax import lax
from jax.experimental import pallas as pl
from jax.experimental.pallas import tpu as pltpu

N_DEV = 16
CAPACITY = 204
CAP_LOCAL = 48

T_BLK = 1024
ROWS = T_BLK // 128
DISP_BLK = 512
N_HOP_R = N_DEV // 2
N_HOP_L = N_DEV - 1 - N_HOP_R


def _ring_allgather_small(r2d):
    m, n = r2d.shape

    def body(x_ref, out_ref, send_sems, recv_sems):
        my = lax.axis_index("i")
        left = (my - 1) % N_DEV
        right = (my + 1) % N_DEV

        barrier = pltpu.get_barrier_semaphore()
        for nbr in (left, right):
            pl.semaphore_signal(barrier, inc=1, device_id=(nbr,),
                                device_id_type=pl.DeviceIdType.MESH)
        pl.semaphore_wait(barrier, 2)

        out_ref[pl.ds(my * m, m), :] = x_ref[:, :]
        for h in range(N_DEV - 1):
            c = (my - h) % N_DEV
            sl = pl.ds(c * m, m)
            rdma = pltpu.make_async_remote_copy(
                src_ref=out_ref.at[sl],
                dst_ref=out_ref.at[sl],
                send_sem=send_sems.at[h],
                recv_sem=recv_sems.at[h],
                device_id=(right,),
                device_id_type=pl.DeviceIdType.MESH,
            )
            rdma.start()
            rdma.wait()

        @functools.partial(pl.run_scoped, sem=pltpu.SemaphoreType.REGULAR)
        def _(sem):
            for nbr in (left, right):
                pl.semaphore_signal(sem, inc=1, device_id=(nbr,),
                                    device_id_type=pl.DeviceIdType.MESH)
            pl.semaphore_wait(sem, 2)

    return pl.pallas_call(
        body,
        out_shape=jax.ShapeDtypeStruct((N_DEV * m, n), r2d.dtype),
        in_specs=[pl.BlockSpec(memory_space=pltpu.VMEM)],
        out_specs=pl.BlockSpec(memory_space=pltpu.VMEM),
        scratch_shapes=[
            pltpu.SemaphoreType.DMA((N_DEV - 1,)),
            pltpu.SemaphoreType.DMA((N_DEV - 1,)),
        ],
        compiler_params=pltpu.CompilerParams(collective_id=0),
    )(r2d)


def _fused_wgather_rank_dispatch(route_g, e2d, x, w):
    n_rows, _ = route_g.shape
    n_glob = n_rows * 128
    m_loc, _ = e2d.shape
    n_tok, d = x.shape
    e_per, _, f = w.shape
    n_exp = N_DEV * e_per
    dump = n_exp * CAP_LOCAL
    n_blocks = n_glob // T_BLK
    blocks_per_dev = n_tok // T_BLK

    def body(route_ref, e_ref, x_ref, w_ref, wout_ref, xg_ref, slot_ref,
             copy_sem, send_r, recv_r, send_l, recv_l,
             acc_buf, pos_buf, counts, counts2):
        my = lax.axis_index("i")
        left = (my - 1) % N_DEV
        right = (my + 1) % N_DEV

        cp = pltpu.make_async_copy(
            w_ref, wout_ref.at[pl.ds(my * e_per, e_per)], copy_sem)
        cp.start()

        barrier = pltpu.get_barrier_semaphore()
        for nbr in (left, right):
            pl.semaphore_signal(barrier, inc=1, device_id=(nbr,),
                                device_id_type=pl.DeviceIdType.MESH)
        pl.semaphore_wait(barrier, 2)
        cp.wait()

        r_iota = lax.broadcasted_iota(jnp.int32, (T_BLK, T_BLK), 0)
        c_iota = lax.broadcasted_iota(jnp.int32, (T_BLK, T_BLK), 1)
        tril = (r_iota >= c_iota).astype(jnp.bfloat16)

        def rank_block(b):
            if b % blocks_per_dev == 0:
                counts2[0, :] = jnp.zeros((n_exp,), jnp.float32)
            route3 = route_ref[pl.ds(b * ROWS, ROWS), :][:, :, None]
            e_iota = lax.broadcasted_iota(jnp.int32, (ROWS, 128, n_exp), 2)
            oh3 = (route3 == e_iota).astype(jnp.float32)
            oh_b = oh3.reshape(T_BLK, n_exp).astype(jnp.bfloat16)
            cs = (jnp.dot(tril, oh_b, preferred_element_type=jnp.float32)
                  + counts[0, :][None, :])
            cs3 = cs.reshape(ROWS, 128, n_exp)
            cnt = jnp.sum(cs3 * oh3, axis=2)
            acc = (cnt <= CAPACITY).astype(jnp.float32)
            oh_acc = (oh3 * acc[:, :, None]).reshape(T_BLK, n_exp).astype(
                jnp.bfloat16)
            cs2 = (jnp.dot(tril, oh_acc, preferred_element_type=jnp.float32)
                   + counts2[0, :][None, :])
            cs2_3 = cs2.reshape(ROWS, 128, n_exp)
            pos = jnp.sum(cs2_3 * oh3, axis=2) - 1.0
            counts[0, :] = cs[T_BLK - 1, :]
            counts2[0, :] = cs2[T_BLK - 1, :]
            acc_buf[pl.ds(b * ROWS, ROWS), :] = acc.astype(jnp.int32)
            pos_buf[pl.ds(b * ROWS, ROWS), :] = pos.astype(jnp.int32)

        def dispatch_phase():
            acc16 = acc_buf[pl.ds(my * m_loc, m_loc), :]
            pos16 = pos_buf[pl.ds(my * m_loc, m_loc), :]
            e16 = e_ref[...]
            valid = (acc16 == 1) & (pos16 < CAP_LOCAL)
            slot16 = jnp.where(valid, e16 * CAP_LOCAL + pos16, dump)
            slot_ref[...] = slot16
            slot_flat = jnp.concatenate(
                [slot16[j:j + 1, :] for j in range(m_loc)], axis=1)
            xb = x_ref[...].astype(jnp.bfloat16)
            for k in range(dump // DISP_BLK):
                s_iota = (lax.broadcasted_iota(
                    jnp.int32, (DISP_BLK, n_tok), 0) + k * DISP_BLK)
                oh = (s_iota == slot_flat).astype(jnp.bfloat16)
                xg_ref[pl.ds(k * DISP_BLK, DISP_BLK), :] = jnp.dot(
                    oh, xb, preferred_element_type=jnp.float32)

        def compute_phase(h):
            if h == 0:
                counts[0, :] = jnp.zeros((n_exp,), jnp.float32)
                for b in range(n_blocks // 2):
                    rank_block(b)
            elif h == 1:
                for b in range(n_blocks // 2, n_blocks):
                    rank_block(b)
            elif h == 2:
                dispatch_phase()

        for h in range(N_HOP_R):
            c_r = (my - h) % N_DEV
            sl_r = pl.ds(c_r * e_per, e_per)
            rdma_r = pltpu.make_async_remote_copy(
                src_ref=wout_ref.at[sl_r],
                dst_ref=wout_ref.at[sl_r],
                send_sem=send_r.at[h],
                recv_sem=recv_r.at[h],
                device_id=(right,),
                device_id_type=pl.DeviceIdType.MESH,
            )
            rdma_r.start()
            if h < N_HOP_L:
                c_l = (my + h) % N_DEV
                sl_l = pl.ds(c_l * e_per, e_per)
                rdma_l = pltpu.make_async_remote_copy(
                    src_ref=wout_ref.at[sl_l],
                    dst_ref=wout_ref.at[sl_l],
                    send_sem=send_l.at[h],
                    recv_sem=recv_l.at[h],
                    device_id=(left,),
                    device_id_type=pl.DeviceIdType.MESH,
                )
                rdma_l.start()
            compute_phase(h)
            rdma_r.wait()
            if h < N_HOP_L:
                rdma_l.wait()

        @functools.partial(pl.run_scoped, sem=pltpu.SemaphoreType.REGULAR)
        def _(sem):
            for nbr in (left, right):
                pl.semaphore_signal(sem, inc=1, device_id=(nbr,),
                                    device_id_type=pl.DeviceIdType.MESH)
            pl.semaphore_wait(sem, 2)

    return pl.pallas_call(
        body,
        out_shape=[
            jax.ShapeDtypeStruct((N_DEV * e_per, d, f), w.dtype),
            jax.ShapeDtypeStruct((dump, d), jnp.float32),
            jax.ShapeDtypeStruct((m_loc, 128), jnp.int32),
        ],
        in_specs=[
            pl.BlockSpec(memory_space=pltpu.VMEM),
            pl.BlockSpec(memory_space=pltpu.VMEM),
            pl.BlockSpec(memory_space=pltpu.VMEM),
            pl.BlockSpec(memory_space=pltpu.MemorySpace.HBM),
        ],
        out_specs=[
            pl.BlockSpec(memory_space=pltpu.MemorySpace.HBM),
            pl.BlockSpec(memory_space=pltpu.VMEM),
            pl.BlockSpec(memory_space=pltpu.VMEM),
        ],
        scratch_shapes=[
            pltpu.SemaphoreType.DMA,
            pltpu.SemaphoreType.DMA((N_HOP_R,)),
            pltpu.SemaphoreType.DMA((N_HOP_R,)),
            pltpu.SemaphoreType.DMA((N_HOP_L,)),
            pltpu.SemaphoreType.DMA((N_HOP_L,)),
            pltpu.VMEM((n_rows, 128), jnp.int32),
            pltpu.VMEM((n_rows, 128), jnp.int32),
            pltpu.VMEM((1, n_exp), jnp.float32),
            pltpu.VMEM((1, n_exp), jnp.float32),
        ],
        compiler_params=pltpu.CompilerParams(collective_id=1),
    )(route_g, e2d, x, w)


def _grouped_matmul(x_grp, w_full):
    s, d = x_grp.shape
    n_exp, _, f = w_full.shape
    cap = s // n_exp

    eb = 2

    def body(x_ref, w_ref, o_ref):
        for j in range(eb):
            o_ref[pl.ds(j * cap, cap), :] = jnp.dot(
                x_ref[pl.ds(j * cap, cap), :].astype(w_ref.dtype), w_ref[j],
                preferred_element_type=jnp.float32)

    return pl.pallas_call(
        body,
        grid=(n_exp // eb,),
        in_specs=[
            pl.BlockSpec((eb * cap, d), lambda e: (e, 0)),
            pl.BlockSpec((eb, d, f), lambda e: (e, 0, 0)),
        ],
        out_specs=pl.BlockSpec((eb * cap, f), lambda e: (e, 0)),
        out_shape=jax.ShapeDtypeStruct((s, f), jnp.float32),
    )(x_grp, w_full)


def _combine_scatter(slot, y_grp, n_tok):
    n_slots, f = y_grp.shape

    def body(s_ref, y_ref, o_ref):
        k = pl.program_id(0)

        @pl.when(k == 0)
        def _():
            o_ref[...] = jnp.zeros_like(o_ref)

        s_iota = (lax.broadcasted_iota(jnp.int32, (DISP_BLK, n_tok), 0)
                  + k * DISP_BLK)
        oh = (s_iota == s_ref[...]).astype(jnp.bfloat16)
        o_ref[...] += lax.dot_general(
            oh, y_ref[...].astype(jnp.bfloat16), (((0,), (0,)), ((), ())),
            preferred_element_type=jnp.float32)

    return pl.pallas_call(
        body,
        grid=(n_slots // DISP_BLK,),
        in_specs=[
            pl.BlockSpec((1, n_tok), lambda k: (0, 0)),
            pl.BlockSpec((DISP_BLK, f), lambda k: (k, 0)),
        ],
        out_specs=pl.BlockSpec((n_tok, f), lambda k: (0, 0)),
        out_shape=jax.ShapeDtypeStruct((n_tok, f), jnp.float32),
    )(slot, y_grp)


def kernel(x, router_W, route_idx, expert_W):
    del router_W
    n_tok, d_model = x.shape
    e_per, _, d_ff = expert_W.shape

    r2d = route_idx.reshape(n_tok // 128, 128)
    route_g = _ring_allgather_small(r2d)

    w_full, x_grp, slot16 = _fused_wgather_rank_dispatch(
        route_g, r2d, x, expert_W.astype(jnp.bfloat16))

    out_grp = _grouped_matmul(x_grp, w_full)
    out = _combine_scatter(slot16.reshape(1, n_tok), out_grp, n_tok)
    return out
